# baseline (speedup 1.0000x reference)
import functools

import jax
import jax.numpy as jnp
from jax import lax
from jax.experimental import pallas as pl
from jax.experimental.pallas import tpu as pltpu
from jax.experimental.pallas import tpu_sc as plsc

N_COMP = 64
ROWS = 16384
NC, NS, LANES = 2, 16, 16
NW = NC * NS
ROWS_W = ROWS // NW
WORDS_W = ROWS_W * N_COMP
GROUPS = ROWS_W // LANES


@functools.partial(
    pl.kernel,
    out_type=(
        jax.ShapeDtypeStruct((NW, N_COMP), jnp.float32),
        jax.ShapeDtypeStruct((NW, LANES), jnp.float32),
    ),
    mesh=plsc.VectorSubcoreMesh(
        core_axis_name="c", subcore_axis_name="s",
        num_cores=NC, num_subcores=NS,
    ),
    scratch_types=(
        pltpu.VMEM((WORDS_W,), jnp.float32),
        pltpu.VMEM((N_COMP,), jnp.float32),
        pltpu.VMEM((LANES,), jnp.float32),
    ),
    compiler_params=pltpu.CompilerParams(needs_layout_passes=False),
)
def _sc_pass(zc_hbm, pop_hbm, conf_hbm, buf, pop, conf):
    wid = lax.axis_index("s") * NC + lax.axis_index("c")
    pltpu.sync_copy(zc_hbm.at[pl.ds(wid * WORDS_W, WORDS_W)], buf)

    zeros16 = jnp.zeros((LANES,), jnp.float32)
    for k in range(N_COMP // LANES):
        pop[pl.ds(k * LANES, LANES)] = zeros16

    row_off = lax.iota(jnp.int32, LANES) * N_COMP
    ones16 = jnp.ones((LANES,), jnp.float32)

    def g_body(g, conf_acc):
        base = row_off + g * (LANES * N_COMP)
        m = jnp.full((LANES,), -1.0, jnp.float32)
        a = jnp.zeros((LANES,), jnp.int32)
        for c in range(N_COMP):
            v = plsc.load_gather(buf, [base + c])
            upd = v > m
            m = jnp.where(upd, v, m)
            a = jnp.where(upd, c, a)
        plsc.store_scatter(pop, [a], ones16)
        return conf_acc + m

    conf_acc = lax.fori_loop(0, GROUPS, g_body, zeros16)
    conf[...] = conf_acc
    pltpu.sync_copy(pop, pop_hbm.at[wid])
    pltpu.sync_copy(conf, conf_hbm.at[wid])


def _merge_body(pop_ref, conf_ref, np_ref, cm_ref):
    present = jnp.max(pop_ref[...], axis=0, keepdims=True)
    num_pop = jnp.sum(jnp.where(present > 0.0, 1.0, 0.0))
    np_ref[...] = num_pop.reshape(1, 1)
    cm_ref[...] = (jnp.sum(conf_ref[...]) * (1.0 / ROWS)).reshape(1, 1)


_merge = pl.pallas_call(
    _merge_body,
    out_shape=(
        jax.ShapeDtypeStruct((1, 1), jnp.float32),
        jax.ShapeDtypeStruct((1, 1), jnp.float32),
    ),
)


def kernel(z, z_cat):
    zc = z_cat.reshape(ROWS * N_COMP)
    pop_part, conf_part = _sc_pass(zc)
    num_pop, conf_mean = _merge(pop_part, conf_part)
    return (z, num_pop[0, 0], conf_mean[0, 0])
